# baseline (speedup 1.0000x reference)
import functools

import jax
import jax.numpy as jnp
from jax import lax
from jax.experimental import pallas as pl
from jax.experimental.pallas import tpu as pltpu
from jax.experimental.pallas import tpu_sc as plsc

NUM_NUMERICAL = 13
N_CAT = 26
CARD = 100000
D_TOKEN = 32
BATCH = 16384
N_TOK = 1 + NUM_NUMERICAL + N_CAT

NW = 32
R = BATCH // NW
C = 64
G = R // C
L = 16


def _sc_body(xnum_hbm, idxT_hbm, w_hbm, nb_hbm, table_hbm, cb_hbm, cls_hbm,
             out_hbm,
             idx_v, rows_v, num_v, xnum_v, w_v, nb_v, cb_v, cls_v,
             sem_g, sem_o):
    wid = lax.axis_index("s") * 2 + lax.axis_index("c")
    base = wid * R

    pltpu.sync_copy(xnum_hbm.at[pl.ds(base, R)], xnum_v)
    pltpu.sync_copy(w_hbm, w_v)
    pltpu.sync_copy(nb_hbm, nb_v)
    pltpu.sync_copy(cb_hbm, cb_v)
    pltpu.sync_copy(cls_hbm, cls_v)

    cls0 = cls_v[pl.ds(0, L)]
    cls1 = cls_v[pl.ds(L, L)]

    def chunk(g, carry):
        row0 = base + g * C

        pltpu.sync_copy(idxT_hbm.at[:, pl.ds(row0, C)], idx_v)

        gathers = []
        for f in range(N_CAT):
            gathers.append(
                pltpu.async_copy(table_hbm.at[idx_v.at[f]], rows_v.at[f],
                                 sem_g))

        def cls_body(r, c):
            num_v[r, 0, pl.ds(0, L)] = cls0
            num_v[r, 0, pl.ds(L, L)] = cls1
            return c
        lax.fori_loop(0, C, cls_body, 0)

        for i in range(NUM_NUMERICAL):
            w0 = w_v[i, pl.ds(0, L)]
            w1 = w_v[i, pl.ds(L, L)]
            b0 = nb_v[i, pl.ds(0, L)]
            b1 = nb_v[i, pl.ds(L, L)]

            def num_body(r, c, i=i, w0=w0, w1=w1, b0=b0, b1=b1):
                xs = xnum_v[g * C + r, :][i]
                num_v[r, 1 + i, pl.ds(0, L)] = xs * w0 + b0
                num_v[r, 1 + i, pl.ds(L, L)] = xs * w1 + b1
                return c
            lax.fori_loop(0, C, num_body, 0)

        for cp in gathers:
            cp.wait()

        for f in range(N_CAT):
            c0 = cb_v[f, pl.ds(0, L)]
            c1 = cb_v[f, pl.ds(L, L)]

            def bias_body(r, c, f=f, c0=c0, c1=c1):
                rows_v[f, r, pl.ds(0, L)] = rows_v[f, r, pl.ds(0, L)] + c0
                rows_v[f, r, pl.ds(L, L)] = rows_v[f, r, pl.ds(L, L)] + c1
                return c
            lax.fori_loop(0, C, bias_body, 0)

        outs = [pltpu.async_copy(
            num_v, out_hbm.at[pl.ds(row0, C), pl.ds(0, 1 + NUM_NUMERICAL), :],
            sem_o)]
        for f in range(N_CAT):
            outs.append(pltpu.async_copy(
                rows_v.at[f], out_hbm.at[pl.ds(row0, C), 1 + NUM_NUMERICAL + f, :],
                sem_o))
        for cp in outs:
            cp.wait()
        return carry

    lax.fori_loop(0, G, chunk, 0)


@jax.jit
def kernel(x_num, x_cat, num_weight, num_bias, table, cat_bias, cls):
    offsets = (jnp.arange(N_CAT, dtype=jnp.int32) * CARD)[:, None]
    idxT = x_cat.astype(jnp.int32).T + offsets
    x_num_p = jnp.pad(x_num, ((0, 0), (0, L - NUM_NUMERICAL)))

    mesh = plsc.VectorSubcoreMesh(core_axis_name="c", subcore_axis_name="s")
    call = pl.kernel(
        _sc_body,
        out_type=jax.ShapeDtypeStruct((BATCH, N_TOK, D_TOKEN), jnp.float32),
        mesh=mesh,
        compiler_params=pltpu.CompilerParams(use_tc_tiling_on_sc=False),
        scratch_types=[
            pltpu.VMEM((N_CAT, C), jnp.int32),
            pltpu.VMEM((N_CAT, C, D_TOKEN), jnp.float32),
            pltpu.VMEM((C, 1 + NUM_NUMERICAL, D_TOKEN), jnp.float32),
            pltpu.VMEM((R, L), jnp.float32),
            pltpu.VMEM((NUM_NUMERICAL, D_TOKEN), jnp.float32),
            pltpu.VMEM((NUM_NUMERICAL, D_TOKEN), jnp.float32),
            pltpu.VMEM((N_CAT, D_TOKEN), jnp.float32),
            pltpu.VMEM((D_TOKEN,), jnp.float32),
            pltpu.SemaphoreType.DMA,
            pltpu.SemaphoreType.DMA,
        ],
    )
    return call(x_num_p, idxT, num_weight, num_bias, table, cat_bias, cls)


# speedup vs baseline: 1.0148x; 1.0148x over previous
import jax
import jax.numpy as jnp
from jax import lax
from jax.experimental import pallas as pl
from jax.experimental.pallas import tpu as pltpu
from jax.experimental.pallas import tpu_sc as plsc

NUM_NUMERICAL = 13
N_CAT = 26
CARD = 100000
D_TOKEN = 32
BATCH = 16384
N_TOK = 1 + NUM_NUMERICAL + N_CAT

NW = 32
R = BATCH // NW
C = 32
G = R // C
L = 16
DB = D_TOKEN // 8
BB = BATCH // 128
TROW = N_TOK * DB


def _sc_body(xnum_hbm, idxT_hbm, w_hbm, nb_hbm, table_hbm, cb_hbm, cls_hbm,
             out_hbm,
             idx_v, rows_v, stage_v, xnum_v, w_v, nb_v, cb_v, cls_v,
             sem_g, sem_o):
    wid = lax.axis_index("s") * 2 + lax.axis_index("c")
    base = wid * R

    pltpu.sync_copy(xnum_hbm.at[pl.ds(base, R)], xnum_v)
    pltpu.sync_copy(w_hbm, w_v)
    pltpu.sync_copy(nb_hbm, nb_v)
    pltpu.sync_copy(cb_hbm, cb_v)
    pltpu.sync_copy(cls_hbm, cls_v)

    cls0 = cls_v[pl.ds(0, L)]
    cls1 = cls_v[pl.ds(L, L)]

    lane = lax.iota(jnp.int32, L)
    db_lo = lane // 8
    db_hi = db_lo + 2
    d_in = lane % 8

    def scat(t, r, v0, v1):
        plsc.store_scatter(stage_v, [t * 4 + db_lo, d_in, jnp.full((L,), r, jnp.int32)], v0)
        plsc.store_scatter(stage_v, [t * 4 + db_hi, d_in, jnp.full((L,), r, jnp.int32)], v1)

    def chunk(g, carry):
        row0 = base + g * C

        pltpu.sync_copy(idxT_hbm.at[:, pl.ds(row0, C)], idx_v)

        gathers = []
        for f in range(N_CAT):
            gathers.append(
                pltpu.async_copy(table_hbm.at[idx_v.at[f]], rows_v.at[f],
                                 sem_g))

        def cls_body(r, c):
            scat(0, r, cls0, cls1)
            return c
        lax.fori_loop(0, C, cls_body, 0)

        for i in range(NUM_NUMERICAL):
            w0 = w_v[i, pl.ds(0, L)]
            w1 = w_v[i, pl.ds(L, L)]
            b0 = nb_v[i, pl.ds(0, L)]
            b1 = nb_v[i, pl.ds(L, L)]

            def num_body(r, c, w0=w0, w1=w1, b0=b0, b1=b1, i=i):
                xs = xnum_v[g * C + r, :][i]
                scat(1 + i, r, xs * w0 + b0, xs * w1 + b1)
                return c
            lax.fori_loop(0, C, num_body, 0)

        for cp in gathers:
            cp.wait()

        for f in range(N_CAT):
            c0 = cb_v[f, pl.ds(0, L)]
            c1 = cb_v[f, pl.ds(L, L)]

            def cat_body(r, c, f=f, c0=c0, c1=c1):
                v0 = rows_v[f, r, pl.ds(0, L)] + c0
                v1 = rows_v[f, r, pl.ds(L, L)] + c1
                scat(1 + NUM_NUMERICAL + f, r, v0, v1)
                return c
            lax.fori_loop(0, C, cat_body, 0)

        bb = wid * (R // 128) + g // (128 // C)
        h = g % (128 // C)
        cp = pltpu.async_copy(
            stage_v, out_hbm.at[:, bb, :, pl.ds(h * C, C)], sem_o)
        cp.wait()
        return carry

    lax.fori_loop(0, G, chunk, 0)


@jax.jit
def kernel(x_num, x_cat, num_weight, num_bias, table, cat_bias, cls):
    offsets = (jnp.arange(N_CAT, dtype=jnp.int32) * CARD)[:, None]
    idxT = x_cat.astype(jnp.int32).T + offsets
    x_num_p = jnp.pad(x_num, ((0, 0), (0, L - NUM_NUMERICAL)))

    mesh = plsc.VectorSubcoreMesh(core_axis_name="c", subcore_axis_name="s")
    call = pl.kernel(
        _sc_body,
        out_type=jax.ShapeDtypeStruct((TROW, BB, 8, 128), jnp.float32),
        mesh=mesh,
        compiler_params=pltpu.CompilerParams(use_tc_tiling_on_sc=False,
                                             needs_layout_passes=False),
        scratch_types=[
            pltpu.VMEM((N_CAT, C), jnp.int32),
            pltpu.VMEM((N_CAT, C, D_TOKEN), jnp.float32),
            pltpu.VMEM((TROW, 8, C), jnp.float32),
            pltpu.VMEM((R, L), jnp.float32),
            pltpu.VMEM((NUM_NUMERICAL, D_TOKEN), jnp.float32),
            pltpu.VMEM((NUM_NUMERICAL, D_TOKEN), jnp.float32),
            pltpu.VMEM((N_CAT, D_TOKEN), jnp.float32),
            pltpu.VMEM((D_TOKEN,), jnp.float32),
            pltpu.SemaphoreType.DMA,
            pltpu.SemaphoreType.DMA,
        ],
    )
    out5 = call(x_num_p, idxT, num_weight, num_bias, table, cat_bias, cls)
    out5 = out5.reshape(N_TOK, DB, BB, 8, 128)
    return out5.transpose(2, 4, 0, 1, 3).reshape(BATCH, N_TOK, D_TOKEN)
